# Initial kernel scaffold; baseline (speedup 1.0000x reference)
import jax
import jax.numpy as jnp
from jax.experimental import pallas as pl


def kernel(x, edge_index, W1, b1, W2, b2, W3, b3):
    raise NotImplementedError("write your pallas kernel here")


# speedup vs baseline: 10.2589x; 10.2589x over previous
import functools

import jax
import jax.numpy as jnp
from jax import lax
from jax.experimental import pallas as pl
from jax.experimental.pallas import tpu as pltpu
from jax.experimental.pallas import tpu_sc as plsc

N = 10000
E = 320000
NP = 10112
NC, NS = 2, 16
NW = NC * NS
CH = 128
NCHUNK = -(-E // (NW * CH))
EW = NCHUNK * CH
EPAD = NW * EW
RPT = NP // NS


def _sc_pass(D, with_gather):
    mesh = plsc.VectorSubcoreMesh(core_axis_name="c", subcore_axis_name="s")

    def body(*refs):
        if with_gather:
            table_hbm, src_hbm, dst_hbm, out_hbm, rows_v, src_v, dst_v, acc_sh = refs
        else:
            src_hbm, dst_hbm, out_hbm, rows_v, src_v, dst_v, acc_sh = refs
        c = lax.axis_index("c")
        s = lax.axis_index("s")
        wid = c * NS + s

        @pl.loop(0, CH)
        def _(i):
            @pl.loop(0, D // 16)
            def _(j):
                rows_v[i, pl.ds(j * 16, 16)] = jnp.zeros((16,), jnp.float32)

        r0 = s * RPT

        @pl.loop(0, RPT // CH)
        def _(k):
            pltpu.sync_copy(rows_v, acc_sh.at[pl.ds(r0 + k * CH, CH)])

        rem = RPT % CH
        if rem:
            pltpu.sync_copy(
                rows_v.at[pl.ds(0, rem)],
                acc_sh.at[pl.ds(r0 + (RPT // CH) * CH, rem)],
            )

        if not with_gather:
            @pl.loop(0, CH)
            def _(i):
                @pl.loop(0, D // 16)
                def _(j):
                    rows_v[i, pl.ds(j * 16, 16)] = jnp.ones((16,), jnp.float32)

        plsc.subcore_barrier()

        base = wid * EW

        @pl.loop(0, NCHUNK)
        def _(t):
            off = base + t * CH
            pltpu.sync_copy(dst_hbm.at[pl.ds(off, CH)], dst_v)
            if with_gather:
                pltpu.sync_copy(src_hbm.at[pl.ds(off, CH)], src_v)
                pltpu.sync_copy(table_hbm.at[src_v], rows_v)
            pltpu.sync_copy(rows_v, acc_sh.at[dst_v], add=True)

        plsc.subcore_barrier()

        @pl.loop(0, RPT // CH)
        def _(k):
            pltpu.sync_copy(
                acc_sh.at[pl.ds(r0 + k * CH, CH)],
                out_hbm.at[c, pl.ds(r0 + k * CH, CH)],
            )

        if rem:
            pltpu.sync_copy(
                acc_sh.at[pl.ds(r0 + (RPT // CH) * CH, rem)],
                out_hbm.at[c, pl.ds(r0 + (RPT // CH) * CH, rem)],
            )

    return pl.kernel(
        body,
        out_type=jax.ShapeDtypeStruct((NC, NP, D), jnp.float32),
        mesh=mesh,
        compiler_params=pltpu.CompilerParams(use_tc_tiling_on_sc=False),
        scratch_types=[
            pltpu.VMEM((CH, D), jnp.float32),
            pltpu.VMEM((CH,), jnp.int32),
            pltpu.VMEM((CH,), jnp.int32),
            pltpu.VMEM_SHARED((NP, D), jnp.float32),
        ],
    )


_DOT = functools.partial(
    lax.dot_general,
    dimension_numbers=(((1,), (0,)), ((), ())),
    precision=lax.Precision.HIGHEST,
    preferred_element_type=jnp.float32,
)


def _dis(deg_ref):
    deg = deg_ref[0, :, 0:1] + deg_ref[1, :, 0:1] + 1.0
    rows = lax.broadcasted_iota(jnp.int32, (NP, 1), 0)
    return jnp.where(rows < N, lax.rsqrt(deg), 0.0)


def _tc_first(deg_ref, x_ref, w_ref, o_ref):
    o_ref[...] = _dis(deg_ref) * _DOT(x_ref[...], w_ref[...])


def _tc_mid(agg_ref, t_ref, deg_ref, w_ref, b_ref, o_ref):
    dis = _dis(deg_ref)
    z = dis * (agg_ref[0] + agg_ref[1] + t_ref[...]) + b_ref[...]
    h = jnp.maximum(z, 0.0)
    o_ref[...] = dis * _DOT(h, w_ref[...])


def _tc_last(agg_ref, t_ref, deg_ref, b_ref, o_ref):
    dis = _dis(deg_ref)[0:N]
    s = agg_ref[0, 0:N, :] + agg_ref[1, 0:N, :] + t_ref[0:N, :]
    o_ref[...] = dis * s + b_ref[...]


def _tc_call(body, out_shape, *args):
    return pl.pallas_call(
        body, out_shape=jax.ShapeDtypeStruct(out_shape, jnp.float32)
    )(*args)


def kernel(x, edge_index, W1, b1, W2, b2, W3, b3):
    ei = edge_index.astype(jnp.int32)
    pad = jnp.full((EPAD - E,), N, jnp.int32)
    src = jnp.concatenate([ei[0], pad])
    dst = jnp.concatenate([ei[1], pad])
    xp = jnp.zeros((NP, x.shape[1]), jnp.float32).at[:N].set(x)

    deg_pl = _sc_pass(16, with_gather=False)(src, dst)

    t1 = _tc_call(_tc_first, (NP, 128), deg_pl, xp, W1)
    agg1 = _sc_pass(128, with_gather=True)(t1, src, dst)

    t2 = _tc_call(_tc_mid, (NP, 128), agg1, t1, deg_pl, W2, b1.reshape(1, -1))
    agg2 = _sc_pass(128, with_gather=True)(t2, src, dst)

    t3 = _tc_call(_tc_mid, (NP, 64), agg2, t2, deg_pl, W3, b2.reshape(1, -1))
    agg3 = _sc_pass(64, with_gather=True)(t3, src, dst)

    return _tc_call(_tc_last, (N, 64), agg3, t3, deg_pl, b3.reshape(1, -1))
